# baseline (speedup 1.0000x reference)
import jax
import jax.numpy as jnp
from jax import lax
from jax.experimental import pallas as pl
from jax.experimental.pallas import tpu as pltpu
from jax.experimental.pallas import tpu_sc as plsc

N = 10000
E = 320000
HID = 128
OUT = 64
EPS = 0.3

NC = 2
NS = 16
NW = NC * NS
BB = 128
NBATCH = 80
EPT = BB * NBATCH
EPAD = EPT * NW
NPAD = 10112
ROWZ = NPAD // NS

_SC_PARAMS = dict(
    compiler_params=pltpu.CompilerParams(needs_layout_passes=False))


def _sc_mesh():
  return plsc.VectorSubcoreMesh(core_axis_name="c", subcore_axis_name="s",
                                num_cores=NC, num_subcores=NS)


NDEG = 10240
RDEG = NDEG // NS


def _deg_body(dst_hbm, degp_hbm, dst_v, degl_v, acc_v, tmp_v, all_sh):
  c = lax.axis_index("c")
  s = lax.axis_index("s")
  wid = c * NS + s
  pltpu.sync_copy(dst_hbm.at[wid], dst_v)

  def zbody(i, carry):
    degl_v[pl.ds(i * 16, 16)] = jnp.zeros((16,), jnp.float32)
    return carry

  lax.fori_loop(0, NDEG // 16, zbody, 0)

  ones16 = jnp.ones((16,), jnp.float32)

  def body(b, carry):
    for g in range(BB // 16):
      dv = dst_v[b, pl.ds(g * 16, 16)]
      plsc.addupdate_scatter(degl_v, [dv], ones16)
    return carry

  lax.fori_loop(0, NBATCH, body, 0)
  pltpu.sync_copy(degl_v, all_sh.at[s])
  plsc.subcore_barrier()

  def z2body(i, carry):
    acc_v[pl.ds(i * 16, 16)] = jnp.zeros((16,), jnp.float32)
    return carry

  lax.fori_loop(0, RDEG // 16, z2body, 0)
  for t in range(NS):
    pltpu.sync_copy(all_sh.at[t, pl.ds(s * RDEG, RDEG)], tmp_v)

    def abody(i, carry):
      sl = pl.ds(i * 16, 16)
      acc_v[sl] = acc_v[sl] + tmp_v[sl]
      return carry

    lax.fori_loop(0, RDEG // 16, abody, 0)
  pltpu.sync_copy(acc_v, degp_hbm.at[c, pl.ds(s * RDEG, RDEG)])


def _make_deg_kernel():
  return pl.kernel(
      _deg_body,
      out_type=jax.ShapeDtypeStruct((NC, NDEG), jnp.float32),
      mesh=_sc_mesh(),
      scratch_types=[
          pltpu.VMEM((NBATCH, BB), jnp.int32),
          pltpu.VMEM((NDEG,), jnp.float32),
          pltpu.VMEM((RDEG,), jnp.float32),
          pltpu.VMEM((RDEG,), jnp.float32),
          pltpu.VMEM_SHARED((NS, NDEG), jnp.float32),
      ],
      **_SC_PARAMS,
  )


def _gate_body(p_hbm, q_hbm, src_hbm, dst_hbm, e_hbm,
               p_v, q_v, src_v, dst_v, e_v):
  c = lax.axis_index("c")
  s = lax.axis_index("s")
  wid = c * NS + s
  pltpu.sync_copy(p_hbm, p_v)
  pltpu.sync_copy(q_hbm, q_v)
  pltpu.sync_copy(src_hbm.at[wid], src_v)
  pltpu.sync_copy(dst_hbm.at[wid], dst_v)

  def ebody(b, carry):
    for g in range(BB // 16):
      sl = pl.ds(g * 16, 16)
      sv = src_v[b, sl]
      dv = dst_v[b, sl]
      u = plsc.load_gather(p_v, [dv]) + plsc.load_gather(q_v, [sv])
      a = jnp.minimum(jnp.abs(u), 20.0)
      e_v[b, sl] = jnp.sign(u) * (1.0 - 2.0 / (jnp.exp(2.0 * a) + 1.0))
    return carry

  lax.fori_loop(0, NBATCH, ebody, 0)
  pltpu.sync_copy(e_v, e_hbm.at[wid])


def _make_gate_kernel():
  return pl.kernel(
      _gate_body,
      out_type=jax.ShapeDtypeStruct((NW, NBATCH, BB), jnp.float32),
      mesh=_sc_mesh(),
      scratch_types=[
          pltpu.VMEM((NPAD,), jnp.float32),
          pltpu.VMEM((NPAD,), jnp.float32),
          pltpu.VMEM((NBATCH, BB), jnp.int32),
          pltpu.VMEM((NBATCH, BB), jnp.int32),
          pltpu.VMEM((NBATCH, BB), jnp.float32),
      ],
      **_SC_PARAMS,
  )


NB0 = 48
NB1 = 112
NBMAX = max(NB0, NB1)
TOTB = NW * NBATCH


def _edge_body(y_hbm, e_hbm, src_hbm, dst_hbm, zeros_hbm,
               zp_hbm, dst_v, sbuf, ebuf, rows0, rows1,
               z_sh, gsem0, gsem1, ssem0, ssem1, isem0, isem1, esem0, esem1):
  c = lax.axis_index("c")
  s = lax.axis_index("s")
  base = jnp.where(c == 0, s * NB0, NS * NB0 + s * NB1)
  nb = jnp.where(c == 0, NB0, NB1)
  pltpu.sync_copy(dst_hbm.at[pl.ds(base, NBMAX)], dst_v)
  pltpu.sync_copy(zeros_hbm, z_sh.at[pl.ds(s * ROWZ, ROWZ)])
  plsc.subcore_barrier()

  gsems = (gsem0, gsem1)
  ssems = (ssem0, ssem1)
  isems = (isem0, isem1)
  esems = (esem0, esem1)
  rows = (rows0, rows1)

  def start_idx(b, j):
    pltpu.async_copy(src_hbm.at[base + b], sbuf.at[j], isems[j])
    pltpu.async_copy(e_hbm.at[base + b], ebuf.at[j], esems[j])

  def wait_idx(b, j):
    pltpu.make_async_copy(src_hbm.at[base + b], sbuf.at[j], isems[j]).wait()
    pltpu.make_async_copy(e_hbm.at[base + b], ebuf.at[j], esems[j]).wait()

  def start_gather(j):
    pltpu.async_copy(y_hbm.at[sbuf.at[j]], rows[j], gsems[j])

  def wait_gather(j):
    pltpu.make_async_copy(y_hbm.at[sbuf.at[j]], rows[j], gsems[j]).wait()

  def scale(j):
    buf = rows[j]

    def rbody(i, carry):
      ev = plsc.load_gather(
          ebuf, [jnp.full((16,), j, jnp.int32), jnp.full((16,), i, jnp.int32)])
      for v in range(HID // 16):
        csl = pl.ds(v * 16, 16)
        buf[i, csl] = buf[i, csl] * ev
      return carry

    lax.fori_loop(0, BB, rbody, 0)

  start_idx(0, 0)
  start_idx(1, 1)
  wait_idx(0, 0)
  start_gather(0)
  wait_idx(1, 1)
  start_gather(1)

  def mbody(k, carry):
    for j in range(2):
      b = 2 * k + j
      wait_gather(j)
      scale(j)
      pltpu.async_copy(rows[j], z_sh.at[dst_v.at[b]], ssems[j], add=True)
    for j in range(2):
      b = 2 * k + j
      nxt = jnp.where(b + 2 >= nb, b + 2 - nb, b + 2)
      pltpu.make_async_copy(rows[j], z_sh.at[dst_v.at[b]], ssems[j]).wait()
      start_idx(nxt, j)
      wait_idx(nxt, j)
      start_gather(j)
    return carry

  lax.fori_loop(0, nb // 2, mbody, 0)
  wait_gather(0)
  wait_gather(1)
  plsc.subcore_barrier()
  pltpu.sync_copy(z_sh.at[pl.ds(s * ROWZ, ROWZ)],
                  zp_hbm.at[c, pl.ds(s * ROWZ, ROWZ)])


def _make_edge_kernel():
  return pl.kernel(
      _edge_body,
      out_type=jax.ShapeDtypeStruct((NC, NPAD, HID), jnp.float32),
      mesh=_sc_mesh(),
      scratch_types=[
          pltpu.VMEM((NBMAX, BB), jnp.int32),
          pltpu.VMEM((2, BB), jnp.int32),
          pltpu.VMEM((2, BB), jnp.float32),
          pltpu.VMEM((BB, HID), jnp.float32),
          pltpu.VMEM((BB, HID), jnp.float32),
          pltpu.VMEM_SHARED((NPAD, HID), jnp.float32),
          pltpu.SemaphoreType.DMA,
          pltpu.SemaphoreType.DMA,
          pltpu.SemaphoreType.DMA,
          pltpu.SemaphoreType.DMA,
          pltpu.SemaphoreType.DMA,
          pltpu.SemaphoreType.DMA,
          pltpu.SemaphoreType.DMA,
          pltpu.SemaphoreType.DMA,
      ],
      **_SC_PARAMS,
  )


def _tc1_body(h_ref, w1_ref, b1_ref, wg_ref, bgv_ref, degp_ref,
              x_ref, pq_ref, d_ref, y_ref):
  x = lax.dot_general(h_ref[...], w1_ref[...], (((1,), (1,)), ((), ())),
                      preferred_element_type=jnp.float32)
  x = jnp.maximum(x + b1_ref[...], 0.0)
  x_ref[...] = x
  pq_ref[...] = lax.dot_general(x, wg_ref[...], (((1,), (1,)), ((), ())),
                                preferred_element_type=jnp.float32) + bgv_ref[...]
  deg = degp_ref[0] + degp_ref[1]
  d = lax.rsqrt(jnp.maximum(deg, 1.0))
  d_ref[...] = d
  y_ref[...] = d * x


def _tc2_body(raw_ref, d_ref, zp_ref, wg_ref, bgv_ref, pq_ref, y_ref):
  d = d_ref[...]
  z = d * (zp_ref[0] + zp_ref[1])
  x = EPS * raw_ref[...] + z
  pq_ref[...] = lax.dot_general(x, wg_ref[...], (((1,), (1,)), ((), ())),
                                preferred_element_type=jnp.float32) + bgv_ref[...]
  y_ref[...] = d * x


def _tc3_body(raw_ref, d_ref, zp_ref, w2_ref, b2_ref, out_ref):
  z = d_ref[...] * (zp_ref[0] + zp_ref[1])
  x = EPS * raw_ref[...] + z
  out_ref[...] = lax.dot_general(x, w2_ref[...], (((1,), (1,)), ((), ())),
                                 preferred_element_type=jnp.float32) + b2_ref[...]


_tc1 = pl.pallas_call(
    _tc1_body,
    out_shape=[
        jax.ShapeDtypeStruct((N, HID), jnp.float32),
        jax.ShapeDtypeStruct((N, 2), jnp.float32),
        jax.ShapeDtypeStruct((N, 1), jnp.float32),
        jax.ShapeDtypeStruct((N, HID), jnp.float32),
    ],
)

_tc2 = pl.pallas_call(
    _tc2_body,
    out_shape=[
        jax.ShapeDtypeStruct((N, 2), jnp.float32),
        jax.ShapeDtypeStruct((N, HID), jnp.float32),
    ],
)

_tc3 = pl.pallas_call(
    _tc3_body,
    out_shape=jax.ShapeDtypeStruct((N, OUT), jnp.float32),
)


def _pad_nodes(v):
  return jnp.concatenate([v, jnp.zeros((NPAD - N,), jnp.float32)])


@jax.jit
def kernel(h, edge_index, W1, b1, W2, b2, Wg1, bg1, Wg2, bg2):
  src = edge_index[0]
  dst = edge_index[1]
  src_p = jnp.concatenate([src, jnp.zeros((EPAD - E,), jnp.int32)])
  dst_p = jnp.concatenate([dst, jnp.full((EPAD - E,), N, jnp.int32)])
  src3 = src_p.reshape(NW, NBATCH, BB)
  dst3 = dst_p.reshape(NW, NBATCH, BB)

  zeros_rows = jnp.zeros((ROWZ, HID), jnp.float32)

  degp = _make_deg_kernel()(dst3)

  wg1 = Wg1.reshape(2, HID)
  wg2 = Wg2.reshape(2, HID)
  bgv1 = jnp.stack([bg1, jnp.zeros((), jnp.float32)]).reshape(1, 2)
  bgv2 = jnp.stack([bg2, jnp.zeros((), jnp.float32)]).reshape(1, 2)

  raw, pq, d, y = _tc1(h, W1, b1.reshape(1, HID), wg1, bgv1,
                       degp[:, :N, None])

  gate = _make_gate_kernel()
  edge = _make_edge_kernel()

  src2 = src_p.reshape(TOTB, BB)
  dst2 = dst_p.reshape(TOTB, BB)

  e1 = gate(_pad_nodes(pq[:, 0]), _pad_nodes(pq[:, 1]), src3, dst3)
  zp = edge(y, e1.reshape(TOTB, BB), src2, dst2, zeros_rows)
  pq2, y2 = _tc2(raw, d, zp[:, :N, :], wg2, bgv2)

  e2 = gate(_pad_nodes(pq2[:, 0]), _pad_nodes(pq2[:, 1]), src3, dst3)
  zp2 = edge(y2, e2.reshape(TOTB, BB), src2, dst2, zeros_rows)
  out = _tc3(raw, d, zp2[:, :N, :], W2, b2.reshape(1, OUT))
  return out


# speedup vs baseline: 1.2125x; 1.2125x over previous
import jax
import jax.numpy as jnp
from jax import lax
from jax.experimental import pallas as pl
from jax.experimental.pallas import tpu as pltpu
from jax.experimental.pallas import tpu_sc as plsc

N = 10000
E = 320000
HID = 128
OUT = 64
EPS = 0.3

NC = 2
NS = 16
NW = NC * NS
BB = 128
NBATCH = 80
EPT = BB * NBATCH
EPAD = EPT * NW
NPAD = 10112
ROWZ = NPAD // NS

_SC_PARAMS = dict(
    compiler_params=pltpu.CompilerParams(needs_layout_passes=False))


def _sc_mesh():
  return plsc.VectorSubcoreMesh(core_axis_name="c", subcore_axis_name="s",
                                num_cores=NC, num_subcores=NS)


NDEG = 10240
RDEG = NDEG // NS


def _deg_body(dst_hbm, degp_hbm, dst_v, degl_v, acc_v, tmp_v, all_sh):
  c = lax.axis_index("c")
  s = lax.axis_index("s")
  wid = c * NS + s
  pltpu.sync_copy(dst_hbm.at[wid], dst_v)

  def zbody(i, carry):
    degl_v[pl.ds(i * 16, 16)] = jnp.zeros((16,), jnp.float32)
    return carry

  lax.fori_loop(0, NDEG // 16, zbody, 0)

  ones16 = jnp.ones((16,), jnp.float32)

  def body(b, carry):
    for g in range(BB // 16):
      dv = dst_v[b, pl.ds(g * 16, 16)]
      plsc.addupdate_scatter(degl_v, [dv], ones16)
    return carry

  lax.fori_loop(0, NBATCH, body, 0)
  pltpu.sync_copy(degl_v, all_sh.at[s])
  plsc.subcore_barrier()

  def z2body(i, carry):
    acc_v[pl.ds(i * 16, 16)] = jnp.zeros((16,), jnp.float32)
    return carry

  lax.fori_loop(0, RDEG // 16, z2body, 0)
  for t in range(NS):
    pltpu.sync_copy(all_sh.at[t, pl.ds(s * RDEG, RDEG)], tmp_v)

    def abody(i, carry):
      sl = pl.ds(i * 16, 16)
      acc_v[sl] = acc_v[sl] + tmp_v[sl]
      return carry

    lax.fori_loop(0, RDEG // 16, abody, 0)
  pltpu.sync_copy(acc_v, degp_hbm.at[c, pl.ds(s * RDEG, RDEG)])


def _make_deg_kernel():
  return pl.kernel(
      _deg_body,
      out_type=jax.ShapeDtypeStruct((NC, NDEG), jnp.float32),
      mesh=_sc_mesh(),
      scratch_types=[
          pltpu.VMEM((NBATCH, BB), jnp.int32),
          pltpu.VMEM((NDEG,), jnp.float32),
          pltpu.VMEM((RDEG,), jnp.float32),
          pltpu.VMEM((RDEG,), jnp.float32),
          pltpu.VMEM_SHARED((NS, NDEG), jnp.float32),
      ],
      **_SC_PARAMS,
  )


def _gate_body(p_hbm, q_hbm, src_hbm, dst_hbm, e_hbm,
               p_v, q_v, src_v, dst_v, e_v):
  c = lax.axis_index("c")
  s = lax.axis_index("s")
  wid = c * NS + s
  pltpu.sync_copy(p_hbm, p_v)
  pltpu.sync_copy(q_hbm, q_v)
  pltpu.sync_copy(src_hbm.at[wid], src_v)
  pltpu.sync_copy(dst_hbm.at[wid], dst_v)

  def ebody(b, carry):
    for g in range(BB // 16):
      sl = pl.ds(g * 16, 16)
      sv = src_v[b, sl]
      dv = dst_v[b, sl]
      u = plsc.load_gather(p_v, [dv]) + plsc.load_gather(q_v, [sv])
      a = jnp.minimum(jnp.abs(u), 20.0)
      e_v[b, sl] = jnp.sign(u) * (1.0 - 2.0 / (jnp.exp(2.0 * a) + 1.0))
    return carry

  lax.fori_loop(0, NBATCH, ebody, 0)
  pltpu.sync_copy(e_v, e_hbm.at[wid])


def _make_gate_kernel():
  return pl.kernel(
      _gate_body,
      out_type=jax.ShapeDtypeStruct((NW, NBATCH, BB), jnp.float32),
      mesh=_sc_mesh(),
      scratch_types=[
          pltpu.VMEM((NPAD,), jnp.float32),
          pltpu.VMEM((NPAD,), jnp.float32),
          pltpu.VMEM((NBATCH, BB), jnp.int32),
          pltpu.VMEM((NBATCH, BB), jnp.int32),
          pltpu.VMEM((NBATCH, BB), jnp.float32),
      ],
      **_SC_PARAMS,
  )


NB0 = 112
NB1 = 48
NBMAX = max(NB0, NB1)
TOTB = NW * NBATCH


def _edge_body(y_hbm, e_hbm, src_hbm, dst_hbm, zeros_hbm,
               zp_hbm, dst_v, sbuf, ebuf, rows0, rows1,
               z_sh, gsem0, gsem1, ssem0, ssem1, isem0, isem1, esem0, esem1):
  c = lax.axis_index("c")
  s = lax.axis_index("s")
  base = jnp.where(c == 0, s * NB0, NS * NB0 + s * NB1)
  nb = jnp.where(c == 0, NB0, NB1)
  pltpu.sync_copy(dst_hbm.at[pl.ds(base, NBMAX)], dst_v)
  pltpu.sync_copy(zeros_hbm, z_sh.at[pl.ds(s * ROWZ, ROWZ)])
  plsc.subcore_barrier()

  gsems = (gsem0, gsem1)
  ssems = (ssem0, ssem1)
  isems = (isem0, isem1)
  esems = (esem0, esem1)
  rows = (rows0, rows1)

  def start_idx(b, j):
    pltpu.async_copy(src_hbm.at[base + b], sbuf.at[j], isems[j])
    pltpu.async_copy(e_hbm.at[base + b], ebuf.at[j], esems[j])

  def wait_idx(b, j):
    pltpu.make_async_copy(src_hbm.at[base + b], sbuf.at[j], isems[j]).wait()
    pltpu.make_async_copy(e_hbm.at[base + b], ebuf.at[j], esems[j]).wait()

  def start_gather(j):
    pltpu.async_copy(y_hbm.at[sbuf.at[j]], rows[j], gsems[j])

  def wait_gather(j):
    pltpu.make_async_copy(y_hbm.at[sbuf.at[j]], rows[j], gsems[j]).wait()

  def scale(j):
    buf = rows[j]

    def rbody(i, carry):
      ev = plsc.load_gather(
          ebuf, [jnp.full((16,), j, jnp.int32), jnp.full((16,), i, jnp.int32)])
      for v in range(HID // 16):
        csl = pl.ds(v * 16, 16)
        buf[i, csl] = buf[i, csl] * ev
      return carry

    lax.fori_loop(0, BB, rbody, 0)

  start_idx(0, 0)
  start_idx(1, 1)
  wait_idx(0, 0)
  start_gather(0)
  wait_idx(1, 1)
  start_gather(1)

  def mbody(k, carry):
    for j in range(2):
      b = 2 * k + j
      wait_gather(j)
      scale(j)
      pltpu.async_copy(rows[j], z_sh.at[dst_v.at[b]], ssems[j], add=True)
    for j in range(2):
      b = 2 * k + j
      nxt = jnp.where(b + 2 >= nb, b + 2 - nb, b + 2)
      pltpu.make_async_copy(rows[j], z_sh.at[dst_v.at[b]], ssems[j]).wait()
      start_idx(nxt, j)
      wait_idx(nxt, j)
      start_gather(j)
    return carry

  lax.fori_loop(0, nb // 2, mbody, 0)
  wait_gather(0)
  wait_gather(1)
  plsc.subcore_barrier()
  pltpu.sync_copy(z_sh.at[pl.ds(s * ROWZ, ROWZ)],
                  zp_hbm.at[c, pl.ds(s * ROWZ, ROWZ)])


def _make_edge_kernel():
  return pl.kernel(
      _edge_body,
      out_type=jax.ShapeDtypeStruct((NC, NPAD, HID), jnp.float32),
      mesh=_sc_mesh(),
      scratch_types=[
          pltpu.VMEM((NBMAX, BB), jnp.int32),
          pltpu.VMEM((2, BB), jnp.int32),
          pltpu.VMEM((2, BB), jnp.float32),
          pltpu.VMEM((BB, HID), jnp.float32),
          pltpu.VMEM((BB, HID), jnp.float32),
          pltpu.VMEM_SHARED((NPAD, HID), jnp.float32),
          pltpu.SemaphoreType.DMA,
          pltpu.SemaphoreType.DMA,
          pltpu.SemaphoreType.DMA,
          pltpu.SemaphoreType.DMA,
          pltpu.SemaphoreType.DMA,
          pltpu.SemaphoreType.DMA,
          pltpu.SemaphoreType.DMA,
          pltpu.SemaphoreType.DMA,
      ],
      **_SC_PARAMS,
  )


def _tc1_body(h_ref, w1_ref, b1_ref, wg_ref, bgv_ref, degp_ref,
              x_ref, pq_ref, d_ref, y_ref):
  x = lax.dot_general(h_ref[...], w1_ref[...], (((1,), (1,)), ((), ())),
                      preferred_element_type=jnp.float32)
  x = jnp.maximum(x + b1_ref[...], 0.0)
  x_ref[...] = x
  pq_ref[...] = lax.dot_general(x, wg_ref[...], (((1,), (1,)), ((), ())),
                                preferred_element_type=jnp.float32) + bgv_ref[...]
  deg = degp_ref[0] + degp_ref[1]
  d = lax.rsqrt(jnp.maximum(deg, 1.0))
  d_ref[...] = d
  y_ref[...] = d * x


def _tc2_body(raw_ref, d_ref, zp_ref, wg_ref, bgv_ref, pq_ref, y_ref):
  d = d_ref[...]
  z = d * (zp_ref[0] + zp_ref[1])
  x = EPS * raw_ref[...] + z
  pq_ref[...] = lax.dot_general(x, wg_ref[...], (((1,), (1,)), ((), ())),
                                preferred_element_type=jnp.float32) + bgv_ref[...]
  y_ref[...] = d * x


def _tc3_body(raw_ref, d_ref, zp_ref, w2_ref, b2_ref, out_ref):
  z = d_ref[...] * (zp_ref[0] + zp_ref[1])
  x = EPS * raw_ref[...] + z
  out_ref[...] = lax.dot_general(x, w2_ref[...], (((1,), (1,)), ((), ())),
                                 preferred_element_type=jnp.float32) + b2_ref[...]


_tc1 = pl.pallas_call(
    _tc1_body,
    out_shape=[
        jax.ShapeDtypeStruct((N, HID), jnp.float32),
        jax.ShapeDtypeStruct((N, 2), jnp.float32),
        jax.ShapeDtypeStruct((N, 1), jnp.float32),
        jax.ShapeDtypeStruct((N, HID), jnp.float32),
    ],
)

_tc2 = pl.pallas_call(
    _tc2_body,
    out_shape=[
        jax.ShapeDtypeStruct((N, 2), jnp.float32),
        jax.ShapeDtypeStruct((N, HID), jnp.float32),
    ],
)

_tc3 = pl.pallas_call(
    _tc3_body,
    out_shape=jax.ShapeDtypeStruct((N, OUT), jnp.float32),
)


def _pad_nodes(v):
  return jnp.concatenate([v, jnp.zeros((NPAD - N,), jnp.float32)])


@jax.jit
def kernel(h, edge_index, W1, b1, W2, b2, Wg1, bg1, Wg2, bg2):
  src = edge_index[0]
  dst = edge_index[1]
  src_p = jnp.concatenate([src, jnp.zeros((EPAD - E,), jnp.int32)])
  dst_p = jnp.concatenate([dst, jnp.full((EPAD - E,), N, jnp.int32)])
  src3 = src_p.reshape(NW, NBATCH, BB)
  dst3 = dst_p.reshape(NW, NBATCH, BB)

  zeros_rows = jnp.zeros((ROWZ, HID), jnp.float32)

  degp = _make_deg_kernel()(dst3)

  wg1 = Wg1.reshape(2, HID)
  wg2 = Wg2.reshape(2, HID)
  bgv1 = jnp.stack([bg1, jnp.zeros((), jnp.float32)]).reshape(1, 2)
  bgv2 = jnp.stack([bg2, jnp.zeros((), jnp.float32)]).reshape(1, 2)

  raw, pq, d, y = _tc1(h, W1, b1.reshape(1, HID), wg1, bgv1,
                       degp[:, :N, None])

  gate = _make_gate_kernel()
  edge = _make_edge_kernel()

  src2 = src_p.reshape(TOTB, BB)
  dst2 = dst_p.reshape(TOTB, BB)

  e1 = gate(_pad_nodes(pq[:, 0]), _pad_nodes(pq[:, 1]), src3, dst3)
  zp = edge(y, e1.reshape(TOTB, BB), src2, dst2, zeros_rows)
  pq2, y2 = _tc2(raw, d, zp[:, :N, :], wg2, bgv2)

  e2 = gate(_pad_nodes(pq2[:, 0]), _pad_nodes(pq2[:, 1]), src3, dst3)
  zp2 = edge(y2, e2.reshape(TOTB, BB), src2, dst2, zeros_rows)
  out = _tc3(raw, d, zp2[:, :N, :], W2, b2.reshape(1, OUT))
  return out


# speedup vs baseline: 1.2573x; 1.0369x over previous
import jax
import jax.numpy as jnp
from jax import lax
from jax.experimental import pallas as pl
from jax.experimental.pallas import tpu as pltpu
from jax.experimental.pallas import tpu_sc as plsc

N = 10000
E = 320000
HID = 128
OUT = 64
EPS = 0.3

NC = 2
NS = 16
NW = NC * NS
BB = 128
NBATCH = 80
EPT = BB * NBATCH
EPAD = EPT * NW
NPAD = 10112
ROWZ = NPAD // NS

_SC_PARAMS = dict(
    compiler_params=pltpu.CompilerParams(needs_layout_passes=False))


def _sc_mesh():
  return plsc.VectorSubcoreMesh(core_axis_name="c", subcore_axis_name="s",
                                num_cores=NC, num_subcores=NS)


NDEG = 10240
RDEG = NDEG // NS


def _deg_body(dst_hbm, degp_hbm, dst_v, degl_v, acc_v, tmp_v, all_sh):
  c = lax.axis_index("c")
  s = lax.axis_index("s")
  wid = c * NS + s
  pltpu.sync_copy(dst_hbm.at[wid], dst_v)

  def zbody(i, carry):
    degl_v[pl.ds(i * 16, 16)] = jnp.zeros((16,), jnp.float32)
    return carry

  lax.fori_loop(0, NDEG // 16, zbody, 0)

  ones16 = jnp.ones((16,), jnp.float32)

  def body(b, carry):
    for g in range(BB // 16):
      dv = dst_v[b, pl.ds(g * 16, 16)]
      plsc.addupdate_scatter(degl_v, [dv], ones16)
    return carry

  lax.fori_loop(0, NBATCH, body, 0)
  pltpu.sync_copy(degl_v, all_sh.at[s])
  plsc.subcore_barrier()

  def z2body(i, carry):
    acc_v[pl.ds(i * 16, 16)] = jnp.zeros((16,), jnp.float32)
    return carry

  lax.fori_loop(0, RDEG // 16, z2body, 0)
  for t in range(NS):
    pltpu.sync_copy(all_sh.at[t, pl.ds(s * RDEG, RDEG)], tmp_v)

    def abody(i, carry):
      sl = pl.ds(i * 16, 16)
      acc_v[sl] = acc_v[sl] + tmp_v[sl]
      return carry

    lax.fori_loop(0, RDEG // 16, abody, 0)
  pltpu.sync_copy(acc_v, degp_hbm.at[c, pl.ds(s * RDEG, RDEG)])


def _make_deg_kernel():
  return pl.kernel(
      _deg_body,
      out_type=jax.ShapeDtypeStruct((NC, NDEG), jnp.float32),
      mesh=_sc_mesh(),
      scratch_types=[
          pltpu.VMEM((NBATCH, BB), jnp.int32),
          pltpu.VMEM((NDEG,), jnp.float32),
          pltpu.VMEM((RDEG,), jnp.float32),
          pltpu.VMEM((RDEG,), jnp.float32),
          pltpu.VMEM_SHARED((NS, NDEG), jnp.float32),
      ],
      **_SC_PARAMS,
  )


def _gate_body(p_hbm, q_hbm, src_hbm, dst_hbm, e_hbm,
               p_v, q_v, src_v, dst_v, e_v):
  c = lax.axis_index("c")
  s = lax.axis_index("s")
  wid = c * NS + s
  pltpu.sync_copy(p_hbm, p_v)
  pltpu.sync_copy(q_hbm, q_v)
  pltpu.sync_copy(src_hbm.at[wid], src_v)
  pltpu.sync_copy(dst_hbm.at[wid], dst_v)

  def ebody(b, carry):
    for g in range(BB // 16):
      sl = pl.ds(g * 16, 16)
      sv = src_v[b, sl]
      dv = dst_v[b, sl]
      u = plsc.load_gather(p_v, [dv]) + plsc.load_gather(q_v, [sv])
      a = jnp.minimum(jnp.abs(u), 20.0)
      e_v[b, sl] = jnp.sign(u) * (1.0 - 2.0 / (jnp.exp(2.0 * a) + 1.0))
    return carry

  lax.fori_loop(0, NBATCH, ebody, 0)
  pltpu.sync_copy(e_v, e_hbm.at[wid])


def _make_gate_kernel():
  return pl.kernel(
      _gate_body,
      out_type=jax.ShapeDtypeStruct((NW, NBATCH, BB), jnp.float32),
      mesh=_sc_mesh(),
      scratch_types=[
          pltpu.VMEM((NPAD,), jnp.float32),
          pltpu.VMEM((NPAD,), jnp.float32),
          pltpu.VMEM((NBATCH, BB), jnp.int32),
          pltpu.VMEM((NBATCH, BB), jnp.int32),
          pltpu.VMEM((NBATCH, BB), jnp.float32),
      ],
      **_SC_PARAMS,
  )


NB0 = 120
NB1 = 40
NBMAX = max(NB0, NB1)
TOTB = NW * NBATCH


def _edge_body(y_hbm, e_hbm, src_hbm, dst_hbm, zeros_hbm,
               zp_hbm, dst_v, sbuf, ebuf, rows0, rows1,
               z_sh, gsem0, gsem1, ssem0, ssem1, isem0, isem1, esem0, esem1):
  c = lax.axis_index("c")
  s = lax.axis_index("s")
  base = jnp.where(c == 0, s * NB0, NS * NB0 + s * NB1)
  nb = jnp.where(c == 0, NB0, NB1)
  pltpu.sync_copy(dst_hbm.at[pl.ds(base, NBMAX)], dst_v)
  pltpu.sync_copy(zeros_hbm, z_sh.at[pl.ds(s * ROWZ, ROWZ)])
  plsc.subcore_barrier()

  gsems = (gsem0, gsem1)
  ssems = (ssem0, ssem1)
  isems = (isem0, isem1)
  esems = (esem0, esem1)
  rows = (rows0, rows1)

  def start_idx(b, j):
    pltpu.async_copy(src_hbm.at[base + b], sbuf.at[j], isems[j])
    pltpu.async_copy(e_hbm.at[base + b], ebuf.at[j], esems[j])

  def wait_idx(b, j):
    pltpu.make_async_copy(src_hbm.at[base + b], sbuf.at[j], isems[j]).wait()
    pltpu.make_async_copy(e_hbm.at[base + b], ebuf.at[j], esems[j]).wait()

  def start_gather(j):
    pltpu.async_copy(y_hbm.at[sbuf.at[j]], rows[j], gsems[j])

  def wait_gather(j):
    pltpu.make_async_copy(y_hbm.at[sbuf.at[j]], rows[j], gsems[j]).wait()

  def scale(j):
    buf = rows[j]

    def rbody(i, carry):
      ev = plsc.load_gather(
          ebuf, [jnp.full((16,), j, jnp.int32), jnp.full((16,), i, jnp.int32)])
      for v in range(HID // 16):
        csl = pl.ds(v * 16, 16)
        buf[i, csl] = buf[i, csl] * ev
      return carry

    lax.fori_loop(0, BB, rbody, 0)

  start_idx(0, 0)
  start_idx(1, 1)
  wait_idx(0, 0)
  start_gather(0)
  wait_idx(1, 1)
  start_gather(1)

  def mbody(k, carry):
    for j in range(2):
      b = 2 * k + j
      wait_gather(j)
      scale(j)
      pltpu.async_copy(rows[j], z_sh.at[dst_v.at[b]], ssems[j], add=True)
    for j in range(2):
      b = 2 * k + j
      nxt = jnp.where(b + 2 >= nb, b + 2 - nb, b + 2)
      pltpu.make_async_copy(rows[j], z_sh.at[dst_v.at[b]], ssems[j]).wait()
      start_idx(nxt, j)
      wait_idx(nxt, j)
      start_gather(j)
    return carry

  lax.fori_loop(0, nb // 2, mbody, 0)
  wait_gather(0)
  wait_gather(1)
  plsc.subcore_barrier()
  pltpu.sync_copy(z_sh.at[pl.ds(s * ROWZ, ROWZ)],
                  zp_hbm.at[c, pl.ds(s * ROWZ, ROWZ)])


def _make_edge_kernel():
  return pl.kernel(
      _edge_body,
      out_type=jax.ShapeDtypeStruct((NC, NPAD, HID), jnp.float32),
      mesh=_sc_mesh(),
      scratch_types=[
          pltpu.VMEM((NBMAX, BB), jnp.int32),
          pltpu.VMEM((2, BB), jnp.int32),
          pltpu.VMEM((2, BB), jnp.float32),
          pltpu.VMEM((BB, HID), jnp.float32),
          pltpu.VMEM((BB, HID), jnp.float32),
          pltpu.VMEM_SHARED((NPAD, HID), jnp.float32),
          pltpu.SemaphoreType.DMA,
          pltpu.SemaphoreType.DMA,
          pltpu.SemaphoreType.DMA,
          pltpu.SemaphoreType.DMA,
          pltpu.SemaphoreType.DMA,
          pltpu.SemaphoreType.DMA,
          pltpu.SemaphoreType.DMA,
          pltpu.SemaphoreType.DMA,
      ],
      **_SC_PARAMS,
  )


def _tc1_body(h_ref, w1_ref, b1_ref, wg_ref, bgv_ref, degp_ref,
              x_ref, pq_ref, d_ref, y_ref):
  x = lax.dot_general(h_ref[...], w1_ref[...], (((1,), (1,)), ((), ())),
                      preferred_element_type=jnp.float32)
  x = jnp.maximum(x + b1_ref[...], 0.0)
  x_ref[...] = x
  pq_ref[...] = lax.dot_general(x, wg_ref[...], (((1,), (1,)), ((), ())),
                                preferred_element_type=jnp.float32) + bgv_ref[...]
  deg = degp_ref[0] + degp_ref[1]
  d = lax.rsqrt(jnp.maximum(deg, 1.0))
  d_ref[...] = d
  y_ref[...] = d * x


def _tc2_body(raw_ref, d_ref, zp_ref, wg_ref, bgv_ref, pq_ref, y_ref):
  d = d_ref[...]
  z = d * (zp_ref[0] + zp_ref[1])
  x = EPS * raw_ref[...] + z
  pq_ref[...] = lax.dot_general(x, wg_ref[...], (((1,), (1,)), ((), ())),
                                preferred_element_type=jnp.float32) + bgv_ref[...]
  y_ref[...] = d * x


def _tc3_body(raw_ref, d_ref, zp_ref, w2_ref, b2_ref, out_ref):
  z = d_ref[...] * (zp_ref[0] + zp_ref[1])
  x = EPS * raw_ref[...] + z
  out_ref[...] = lax.dot_general(x, w2_ref[...], (((1,), (1,)), ((), ())),
                                 preferred_element_type=jnp.float32) + b2_ref[...]


_tc1 = pl.pallas_call(
    _tc1_body,
    out_shape=[
        jax.ShapeDtypeStruct((N, HID), jnp.float32),
        jax.ShapeDtypeStruct((N, 2), jnp.float32),
        jax.ShapeDtypeStruct((N, 1), jnp.float32),
        jax.ShapeDtypeStruct((N, HID), jnp.float32),
    ],
)

_tc2 = pl.pallas_call(
    _tc2_body,
    out_shape=[
        jax.ShapeDtypeStruct((N, 2), jnp.float32),
        jax.ShapeDtypeStruct((N, HID), jnp.float32),
    ],
)

_tc3 = pl.pallas_call(
    _tc3_body,
    out_shape=jax.ShapeDtypeStruct((N, OUT), jnp.float32),
)


def _pad_nodes(v):
  return jnp.concatenate([v, jnp.zeros((NPAD - N,), jnp.float32)])


@jax.jit
def kernel(h, edge_index, W1, b1, W2, b2, Wg1, bg1, Wg2, bg2):
  src = edge_index[0]
  dst = edge_index[1]
  src_p = jnp.concatenate([src, jnp.zeros((EPAD - E,), jnp.int32)])
  dst_p = jnp.concatenate([dst, jnp.full((EPAD - E,), N, jnp.int32)])
  src3 = src_p.reshape(NW, NBATCH, BB)
  dst3 = dst_p.reshape(NW, NBATCH, BB)

  zeros_rows = jnp.zeros((ROWZ, HID), jnp.float32)

  degp = _make_deg_kernel()(dst3)

  wg1 = Wg1.reshape(2, HID)
  wg2 = Wg2.reshape(2, HID)
  bgv1 = jnp.stack([bg1, jnp.zeros((), jnp.float32)]).reshape(1, 2)
  bgv2 = jnp.stack([bg2, jnp.zeros((), jnp.float32)]).reshape(1, 2)

  raw, pq, d, y = _tc1(h, W1, b1.reshape(1, HID), wg1, bgv1,
                       degp[:, :N, None])

  gate = _make_gate_kernel()
  edge = _make_edge_kernel()

  src2 = src_p.reshape(TOTB, BB)
  dst2 = dst_p.reshape(TOTB, BB)

  e1 = gate(_pad_nodes(pq[:, 0]), _pad_nodes(pq[:, 1]), src3, dst3)
  zp = edge(y, e1.reshape(TOTB, BB), src2, dst2, zeros_rows)
  pq2, y2 = _tc2(raw, d, zp[:, :N, :], wg2, bgv2)

  e2 = gate(_pad_nodes(pq2[:, 0]), _pad_nodes(pq2[:, 1]), src3, dst3)
  zp2 = edge(y2, e2.reshape(TOTB, BB), src2, dst2, zeros_rows)
  out = _tc3(raw, d, zp2[:, :N, :], W2, b2.reshape(1, OUT))
  return out


# speedup vs baseline: 1.2973x; 1.0318x over previous
import jax
import jax.numpy as jnp
from jax import lax
from jax.experimental import pallas as pl
from jax.experimental.pallas import tpu as pltpu
from jax.experimental.pallas import tpu_sc as plsc

N = 10000
E = 320000
HID = 128
OUT = 64
EPS = 0.3

NC = 2
NS = 16
NW = NC * NS
BB = 128
NBATCH = 80
EPT = BB * NBATCH
EPAD = EPT * NW
NPAD = 10112
ROWZ = NPAD // NS

_SC_PARAMS = dict(
    compiler_params=pltpu.CompilerParams(needs_layout_passes=False))


def _sc_mesh():
  return plsc.VectorSubcoreMesh(core_axis_name="c", subcore_axis_name="s",
                                num_cores=NC, num_subcores=NS)


NDEG = 10240
RDEG = NDEG // NS


def _deg_body(dst_hbm, degp_hbm, dst_v, degl_v, acc_v, tmp_v, all_sh):
  c = lax.axis_index("c")
  s = lax.axis_index("s")
  wid = c * NS + s
  pltpu.sync_copy(dst_hbm.at[wid], dst_v)

  def zbody(i, carry):
    degl_v[pl.ds(i * 16, 16)] = jnp.zeros((16,), jnp.float32)
    return carry

  lax.fori_loop(0, NDEG // 16, zbody, 0)

  ones16 = jnp.ones((16,), jnp.float32)

  def body(b, carry):
    for g in range(BB // 16):
      dv = dst_v[b, pl.ds(g * 16, 16)]
      plsc.addupdate_scatter(degl_v, [dv], ones16)
    return carry

  lax.fori_loop(0, NBATCH, body, 0)
  pltpu.sync_copy(degl_v, all_sh.at[s])
  plsc.subcore_barrier()

  def z2body(i, carry):
    acc_v[pl.ds(i * 16, 16)] = jnp.zeros((16,), jnp.float32)
    return carry

  lax.fori_loop(0, RDEG // 16, z2body, 0)
  for t in range(NS):
    pltpu.sync_copy(all_sh.at[t, pl.ds(s * RDEG, RDEG)], tmp_v)

    def abody(i, carry):
      sl = pl.ds(i * 16, 16)
      acc_v[sl] = acc_v[sl] + tmp_v[sl]
      return carry

    lax.fori_loop(0, RDEG // 16, abody, 0)
  pltpu.sync_copy(acc_v, degp_hbm.at[c, pl.ds(s * RDEG, RDEG)])


def _make_deg_kernel():
  return pl.kernel(
      _deg_body,
      out_type=jax.ShapeDtypeStruct((NC, NDEG), jnp.float32),
      mesh=_sc_mesh(),
      scratch_types=[
          pltpu.VMEM((NBATCH, BB), jnp.int32),
          pltpu.VMEM((NDEG,), jnp.float32),
          pltpu.VMEM((RDEG,), jnp.float32),
          pltpu.VMEM((RDEG,), jnp.float32),
          pltpu.VMEM_SHARED((NS, NDEG), jnp.float32),
      ],
      **_SC_PARAMS,
  )


def _gate_body(p_hbm, q_hbm, src_hbm, dst_hbm, e_hbm,
               p_v, q_v, src_v, dst_v, e_v):
  c = lax.axis_index("c")
  s = lax.axis_index("s")
  wid = c * NS + s
  pltpu.sync_copy(p_hbm, p_v)
  pltpu.sync_copy(q_hbm, q_v)
  pltpu.sync_copy(src_hbm.at[wid], src_v)
  pltpu.sync_copy(dst_hbm.at[wid], dst_v)

  def ebody(b, carry):
    for g in range(BB // 16):
      sl = pl.ds(g * 16, 16)
      sv = src_v[b, sl]
      dv = dst_v[b, sl]
      u = plsc.load_gather(p_v, [dv]) + plsc.load_gather(q_v, [sv])
      a = jnp.minimum(jnp.abs(u), 20.0)
      e_v[b, sl] = jnp.sign(u) * (1.0 - 2.0 / (jnp.exp(2.0 * a) + 1.0))
    return carry

  lax.fori_loop(0, NBATCH, ebody, 0)
  pltpu.sync_copy(e_v, e_hbm.at[wid])


def _make_gate_kernel():
  return pl.kernel(
      _gate_body,
      out_type=jax.ShapeDtypeStruct((NW, NBATCH, BB), jnp.float32),
      mesh=_sc_mesh(),
      scratch_types=[
          pltpu.VMEM((NPAD,), jnp.float32),
          pltpu.VMEM((NPAD,), jnp.float32),
          pltpu.VMEM((NBATCH, BB), jnp.int32),
          pltpu.VMEM((NBATCH, BB), jnp.int32),
          pltpu.VMEM((NBATCH, BB), jnp.float32),
      ],
      **_SC_PARAMS,
  )


NB0 = 128
NB1 = 32
NBMAX = max(NB0, NB1)
TOTB = NW * NBATCH


def _edge_body(y_hbm, e_hbm, src_hbm, dst_hbm, zeros_hbm,
               zp_hbm, dst_v, sbuf, ebuf, rows0, rows1,
               z_sh, gsem0, gsem1, ssem0, ssem1, isem0, isem1, esem0, esem1):
  c = lax.axis_index("c")
  s = lax.axis_index("s")
  base = jnp.where(c == 0, s * NB0, NS * NB0 + s * NB1)
  nb = jnp.where(c == 0, NB0, NB1)
  pltpu.sync_copy(dst_hbm.at[pl.ds(base, NBMAX)], dst_v)
  pltpu.sync_copy(zeros_hbm, z_sh.at[pl.ds(s * ROWZ, ROWZ)])
  plsc.subcore_barrier()

  gsems = (gsem0, gsem1)
  ssems = (ssem0, ssem1)
  isems = (isem0, isem1)
  esems = (esem0, esem1)
  rows = (rows0, rows1)

  def start_idx(b, j):
    pltpu.async_copy(src_hbm.at[base + b], sbuf.at[j], isems[j])
    pltpu.async_copy(e_hbm.at[base + b], ebuf.at[j], esems[j])

  def wait_idx(b, j):
    pltpu.make_async_copy(src_hbm.at[base + b], sbuf.at[j], isems[j]).wait()
    pltpu.make_async_copy(e_hbm.at[base + b], ebuf.at[j], esems[j]).wait()

  def start_gather(j):
    pltpu.async_copy(y_hbm.at[sbuf.at[j]], rows[j], gsems[j])

  def wait_gather(j):
    pltpu.make_async_copy(y_hbm.at[sbuf.at[j]], rows[j], gsems[j]).wait()

  def scale(j):
    buf = rows[j]

    def rbody(i, carry):
      ev = plsc.load_gather(
          ebuf, [jnp.full((16,), j, jnp.int32), jnp.full((16,), i, jnp.int32)])
      for v in range(HID // 16):
        csl = pl.ds(v * 16, 16)
        buf[i, csl] = buf[i, csl] * ev
      return carry

    lax.fori_loop(0, BB, rbody, 0)

  start_idx(0, 0)
  start_idx(1, 1)
  wait_idx(0, 0)
  start_gather(0)
  wait_idx(1, 1)
  start_gather(1)

  def mbody(k, carry):
    for j in range(2):
      b = 2 * k + j
      wait_gather(j)
      scale(j)
      pltpu.async_copy(rows[j], z_sh.at[dst_v.at[b]], ssems[j], add=True)
    for j in range(2):
      b = 2 * k + j
      nxt = jnp.where(b + 2 >= nb, b + 2 - nb, b + 2)
      pltpu.make_async_copy(rows[j], z_sh.at[dst_v.at[b]], ssems[j]).wait()
      start_idx(nxt, j)
      wait_idx(nxt, j)
      start_gather(j)
    return carry

  lax.fori_loop(0, nb // 2, mbody, 0)
  wait_gather(0)
  wait_gather(1)
  plsc.subcore_barrier()
  pltpu.sync_copy(z_sh.at[pl.ds(s * ROWZ, ROWZ)],
                  zp_hbm.at[c, pl.ds(s * ROWZ, ROWZ)])


def _make_edge_kernel():
  return pl.kernel(
      _edge_body,
      out_type=jax.ShapeDtypeStruct((NC, NPAD, HID), jnp.float32),
      mesh=_sc_mesh(),
      scratch_types=[
          pltpu.VMEM((NBMAX, BB), jnp.int32),
          pltpu.VMEM((2, BB), jnp.int32),
          pltpu.VMEM((2, BB), jnp.float32),
          pltpu.VMEM((BB, HID), jnp.float32),
          pltpu.VMEM((BB, HID), jnp.float32),
          pltpu.VMEM_SHARED((NPAD, HID), jnp.float32),
          pltpu.SemaphoreType.DMA,
          pltpu.SemaphoreType.DMA,
          pltpu.SemaphoreType.DMA,
          pltpu.SemaphoreType.DMA,
          pltpu.SemaphoreType.DMA,
          pltpu.SemaphoreType.DMA,
          pltpu.SemaphoreType.DMA,
          pltpu.SemaphoreType.DMA,
      ],
      **_SC_PARAMS,
  )


def _tc1_body(h_ref, w1_ref, b1_ref, wg_ref, bgv_ref, degp_ref,
              x_ref, pq_ref, d_ref, y_ref):
  x = lax.dot_general(h_ref[...], w1_ref[...], (((1,), (1,)), ((), ())),
                      preferred_element_type=jnp.float32)
  x = jnp.maximum(x + b1_ref[...], 0.0)
  x_ref[...] = x
  pq_ref[...] = lax.dot_general(x, wg_ref[...], (((1,), (1,)), ((), ())),
                                preferred_element_type=jnp.float32) + bgv_ref[...]
  deg = degp_ref[0] + degp_ref[1]
  d = lax.rsqrt(jnp.maximum(deg, 1.0))
  d_ref[...] = d
  y_ref[...] = d * x


def _tc2_body(raw_ref, d_ref, zp_ref, wg_ref, bgv_ref, pq_ref, y_ref):
  d = d_ref[...]
  z = d * (zp_ref[0] + zp_ref[1])
  x = EPS * raw_ref[...] + z
  pq_ref[...] = lax.dot_general(x, wg_ref[...], (((1,), (1,)), ((), ())),
                                preferred_element_type=jnp.float32) + bgv_ref[...]
  y_ref[...] = d * x


def _tc3_body(raw_ref, d_ref, zp_ref, w2_ref, b2_ref, out_ref):
  z = d_ref[...] * (zp_ref[0] + zp_ref[1])
  x = EPS * raw_ref[...] + z
  out_ref[...] = lax.dot_general(x, w2_ref[...], (((1,), (1,)), ((), ())),
                                 preferred_element_type=jnp.float32) + b2_ref[...]


_tc1 = pl.pallas_call(
    _tc1_body,
    out_shape=[
        jax.ShapeDtypeStruct((N, HID), jnp.float32),
        jax.ShapeDtypeStruct((N, 2), jnp.float32),
        jax.ShapeDtypeStruct((N, 1), jnp.float32),
        jax.ShapeDtypeStruct((N, HID), jnp.float32),
    ],
)

_tc2 = pl.pallas_call(
    _tc2_body,
    out_shape=[
        jax.ShapeDtypeStruct((N, 2), jnp.float32),
        jax.ShapeDtypeStruct((N, HID), jnp.float32),
    ],
)

_tc3 = pl.pallas_call(
    _tc3_body,
    out_shape=jax.ShapeDtypeStruct((N, OUT), jnp.float32),
)


def _pad_nodes(v):
  return jnp.concatenate([v, jnp.zeros((NPAD - N,), jnp.float32)])


@jax.jit
def kernel(h, edge_index, W1, b1, W2, b2, Wg1, bg1, Wg2, bg2):
  src = edge_index[0]
  dst = edge_index[1]
  src_p = jnp.concatenate([src, jnp.zeros((EPAD - E,), jnp.int32)])
  dst_p = jnp.concatenate([dst, jnp.full((EPAD - E,), N, jnp.int32)])
  src3 = src_p.reshape(NW, NBATCH, BB)
  dst3 = dst_p.reshape(NW, NBATCH, BB)

  zeros_rows = jnp.zeros((ROWZ, HID), jnp.float32)

  degp = _make_deg_kernel()(dst3)

  wg1 = Wg1.reshape(2, HID)
  wg2 = Wg2.reshape(2, HID)
  bgv1 = jnp.stack([bg1, jnp.zeros((), jnp.float32)]).reshape(1, 2)
  bgv2 = jnp.stack([bg2, jnp.zeros((), jnp.float32)]).reshape(1, 2)

  raw, pq, d, y = _tc1(h, W1, b1.reshape(1, HID), wg1, bgv1,
                       degp[:, :N, None])

  gate = _make_gate_kernel()
  edge = _make_edge_kernel()

  src2 = src_p.reshape(TOTB, BB)
  dst2 = dst_p.reshape(TOTB, BB)

  e1 = gate(_pad_nodes(pq[:, 0]), _pad_nodes(pq[:, 1]), src3, dst3)
  zp = edge(y, e1.reshape(TOTB, BB), src2, dst2, zeros_rows)
  pq2, y2 = _tc2(raw, d, zp[:, :N, :], wg2, bgv2)

  e2 = gate(_pad_nodes(pq2[:, 0]), _pad_nodes(pq2[:, 1]), src3, dst3)
  zp2 = edge(y2, e2.reshape(TOTB, BB), src2, dst2, zeros_rows)
  out = _tc3(raw, d, zp2[:, :N, :], W2, b2.reshape(1, OUT))
  return out
